# Initial kernel scaffold; baseline (speedup 1.0000x reference)
import jax
import jax.numpy as jnp
from jax.experimental import pallas as pl


def kernel(input_x, support_nodes_layer1, support_nodes_layer2, support_sessions_layer1, support_sessions_layer2, item_emb, user_emb, W1, W2, gat_w, gat_b):
    raise NotImplementedError("write your pallas kernel here")


# speedup vs baseline: 3.2495x; 3.2495x over previous
import functools

import jax
import jax.numpy as jnp
from jax import lax
from jax.experimental import pallas as pl
from jax.experimental.pallas import tpu as pltpu
from jax.experimental.pallas import tpu_sc as plsc

NC, NS, LANES = 2, 16, 16
NW = NC * NS
D = 100
DP = 128
L = 20
S1 = 10
S2 = 5
B = 1024


@functools.lru_cache(maxsize=None)
def _make_sess_kernel(n_rows: int, chunk: int, with_user: bool):
    per_w = n_rows // NW
    steps = per_w // chunk
    assert n_rows == NW * steps * chunk, (n_rows, chunk)
    C = chunk
    mesh = plsc.VectorSubcoreMesh(core_axis_name="c", subcore_axis_name="s")

    st_ty = jax.ShapeDtypeStruct((n_rows, DP), jnp.float32)
    lt_ty = jax.ShapeDtypeStruct((n_rows, DP), jnp.float32)
    out_type = (st_ty, lt_ty) if with_user else st_ty

    scratch = [
        pltpu.VMEM((C * L,), jnp.int32),
        pltpu.VMEM((C * L, DP), jnp.float32),
        pltpu.VMEM((C, DP), jnp.float32),
        pltpu.SemaphoreType.DMA,
    ]
    if with_user:
        scratch += [
            pltpu.VMEM((C,), jnp.int32),
            pltpu.VMEM((C, DP), jnp.float32),
            pltpu.SemaphoreType.DMA,
        ]

    def body(*refs):
        if with_user:
            (item_hbm, user_hbm, sess_hbm, nodes_hbm, st_out, lt_out,
             idx_v, rows_v, out_v, sem, uidx_v, urows_v, usem) = refs
        else:
            (item_hbm, sess_hbm, st_out,
             idx_v, rows_v, out_v, sem) = refs
        w = lax.axis_index("s") * NC + lax.axis_index("c")

        def step(i, carry):
            base = (w * steps + i) * C
            pltpu.sync_copy(sess_hbm.at[pl.ds(base * L, C * L)], idx_v)
            gat = pltpu.make_async_copy(item_hbm.at[idx_v], rows_v, sem)
            gat.start()
            if with_user:
                pltpu.sync_copy(nodes_hbm.at[pl.ds(base, C)], uidx_v)
                ugat = pltpu.make_async_copy(user_hbm.at[uidx_v], urows_v, usem)
                ugat.start()
            gat.wait()

            def sess(s, c2):
                for k in range(DP // 16):
                    off = k * 16
                    a = rows_v[s * L, pl.ds(off, 16)]
                    for l in range(1, L):
                        a = a + rows_v[s * L + l, pl.ds(off, 16)]
                    out_v[s, pl.ds(off, 16)] = a
                return c2

            lax.fori_loop(0, C, sess, 0)
            pltpu.sync_copy(out_v, st_out.at[pl.ds(base, C)])
            if with_user:
                ugat.wait()
                pltpu.sync_copy(urows_v, lt_out.at[pl.ds(base, C)])
            return carry

        lax.fori_loop(0, steps, step, 0)

    return pl.kernel(body, out_type=out_type, mesh=mesh, scratch_types=scratch)


def _masked_mean(rs, sess, e0):
    cnt0 = jnp.sum((sess == 0).astype(jnp.float32), axis=1, keepdims=True)
    den = jnp.maximum(jnp.float32(L) - cnt0, 1.0)
    return (rs[:, :D] - cnt0 * e0) / den


def _h_block(lt_ref, rs_ref, sess_ref, wa_ref, wb_ref, e0_ref, o_ref):
    st = _masked_mean(rs_ref[...], sess_ref[...], e0_ref[...])
    x = jnp.dot(lt_ref[:, :D], wa_ref[...], preferred_element_type=jnp.float32)
    x = x + jnp.dot(st, wb_ref[...], preferred_element_type=jnp.float32)
    o_ref[...] = jnp.tanh(x)


def _h_layer(lt, rs, sess, wa, wb, e0, bs):
    n = lt.shape[0]
    return pl.pallas_call(
        _h_block,
        grid=(n // bs,),
        in_specs=[
            pl.BlockSpec((bs, DP), lambda i: (i, 0)),
            pl.BlockSpec((bs, DP), lambda i: (i, 0)),
            pl.BlockSpec((bs, L), lambda i: (i, 0)),
            pl.BlockSpec((D, D), lambda i: (0, 0)),
            pl.BlockSpec((D, D), lambda i: (0, 0)),
            pl.BlockSpec((1, D), lambda i: (0, 0)),
        ],
        out_specs=pl.BlockSpec((bs, D), lambda i: (i, 0)),
        out_shape=jax.ShapeDtypeStruct((n, D), jnp.float32),
    )(lt, rs, sess, wa, wb, e0)


def _gat_math(selfv, neigh, k, wt, b):
    sn = jnp.sum(neigh * selfv[:, None, :], axis=2)
    ss = jnp.sum(selfv * selfv, axis=1, keepdims=True)
    s = jnp.concatenate([sn, ss], axis=1)
    m = jnp.max(s, axis=1, keepdims=True)
    e = jnp.exp(s - m)
    a = e / jnp.sum(e, axis=1, keepdims=True)
    ctx = jnp.sum(neigh * a[:, :k, None], axis=1) + selfv * a[:, k:k + 1]
    return jnp.maximum(
        jnp.dot(ctx, wt, preferred_element_type=jnp.float32) + b, 0.0)


def _gat0_block(h2_ref, h1_ref, w_ref, b_ref, o_ref):
    bs = h2_ref.shape[0]
    neigh = h1_ref[...].reshape(bs, S1, D)
    o_ref[...] = _gat_math(h2_ref[...], neigh, S1, w_ref[...], b_ref[...])


def _gat0(h2, h1, wt, b, bs):
    n = h2.shape[0]
    return pl.pallas_call(
        _gat0_block,
        grid=(n // bs,),
        in_specs=[
            pl.BlockSpec((bs, D), lambda i: (i, 0)),
            pl.BlockSpec((bs * S1, D), lambda i: (i, 0)),
            pl.BlockSpec((D, D), lambda i: (0, 0)),
            pl.BlockSpec((1, D), lambda i: (0, 0)),
        ],
        out_specs=pl.BlockSpec((bs, D), lambda i: (i, 0)),
        out_shape=jax.ShapeDtypeStruct((n, D), jnp.float32),
    )(h2, h1, wt, b)


def _gat1_feat_block(hu_rs_ref, ix_ref, h2a_ref, w_ref, b_ref,
                     w2a_ref, w2b_ref, e0_ref, o_ref):
    hu = _masked_mean(hu_rs_ref[...], ix_ref[...], e0_ref[...])
    neigh = h2a_ref[...].reshape(B, S2, D)
    soc = _gat_math(hu, neigh, S2, w_ref[...], b_ref[...])
    o_ref[...] = (
        jnp.dot(hu, w2a_ref[...], preferred_element_type=jnp.float32)
        + jnp.dot(soc, w2b_ref[...], preferred_element_type=jnp.float32))


def _gat1_feat(hu_rs, ix, h2a, wt, b, w2a, w2b, e0):
    return pl.pallas_call(
        _gat1_feat_block,
        out_shape=jax.ShapeDtypeStruct((B, D), jnp.float32),
    )(hu_rs, ix, h2a, wt, b, w2a, w2b, e0)


def _logits_block(feat_ref, it_ref, o_ref):
    o_ref[...] = lax.dot_general(
        feat_ref[...], it_ref[...], (((1,), (1,)), ((), ())),
        preferred_element_type=jnp.float32)


def _logits(feat, item_emb, vb):
    nv = item_emb.shape[0]
    return pl.pallas_call(
        _logits_block,
        grid=(pl.cdiv(nv, vb),),
        in_specs=[
            pl.BlockSpec((B, D), lambda i: (0, 0)),
            pl.BlockSpec((vb, D), lambda i: (i, 0)),
        ],
        out_specs=pl.BlockSpec((B, vb), lambda i: (0, i)),
        out_shape=jax.ShapeDtypeStruct((B, nv), jnp.float32),
    )(feat, item_emb)


def kernel(input_x, support_nodes_layer1, support_nodes_layer2,
           support_sessions_layer1, support_sessions_layer2,
           item_emb, user_emb, W1, W2, gat_w, gat_b):
    input_x = jnp.asarray(input_x, jnp.int32)
    sn1 = jnp.asarray(support_nodes_layer1, jnp.int32)
    sn2 = jnp.asarray(support_nodes_layer2, jnp.int32)
    ss1 = jnp.asarray(support_sessions_layer1, jnp.int32)
    ss2 = jnp.asarray(support_sessions_layer2, jnp.int32)

    item_pad = jnp.pad(item_emb, ((0, 0), (0, DP - D)))
    user_pad = jnp.pad(user_emb, ((0, 0), (0, DP - D)))
    e0 = item_emb[0:1, :]

    hu_rs = _make_sess_kernel(B, 16, False)(item_pad, input_x.reshape(-1))
    rs1, lt1 = _make_sess_kernel(B * S2 * S1, 16, True)(
        item_pad, user_pad, ss1.reshape(-1), sn1)
    rs2, lt2 = _make_sess_kernel(B * S2, 16, True)(
        item_pad, user_pad, ss2.reshape(-1), sn2)

    w1a = W1[:, :D].T
    w1b = W1[:, D:].T
    w0t = gat_w[0].T
    w1t = gat_w[1].T
    b0 = gat_b[0].reshape(1, D)
    b1 = gat_b[1].reshape(1, D)
    w2a = W2[:, :D].T
    w2b = W2[:, D:].T

    h1 = _h_layer(lt1, rs1, ss1, w1a, w1b, e0, 512)
    h2 = _h_layer(lt2, rs2, ss2, w1a, w1b, e0, 512)
    h2_agg = _gat0(h2, h1, w0t, b0, 512)
    feat = _gat1_feat(hu_rs, input_x, h2_agg, w1t, b1, w2a, w2b, e0)
    return _logits(feat, item_emb, 2048)


# speedup vs baseline: 3.7429x; 1.1519x over previous
import functools

import jax
import jax.numpy as jnp
from jax import lax
from jax.experimental import pallas as pl
from jax.experimental.pallas import tpu as pltpu
from jax.experimental.pallas import tpu_sc as plsc

NC, NS, LANES = 2, 16, 16
NW = NC * NS
D = 100
DP = 128
L = 20
S1 = 10
S2 = 5
B = 1024


@functools.lru_cache(maxsize=None)
def _make_sess_kernel(n_rows: int, chunk: int, with_user: bool):
    per_w = n_rows // NW
    steps = per_w // chunk
    assert n_rows == NW * steps * chunk and steps % 2 == 0, (n_rows, chunk)
    C = chunk
    mesh = plsc.VectorSubcoreMesh(core_axis_name="c", subcore_axis_name="s")

    st_ty = jax.ShapeDtypeStruct((n_rows, DP), jnp.float32)
    lt_ty = jax.ShapeDtypeStruct((n_rows, DP), jnp.float32)
    out_type = (st_ty, lt_ty) if with_user else st_ty

    scratch = [
        pltpu.VMEM((C * L,), jnp.int32),
        pltpu.VMEM((C * L,), jnp.int32),
        pltpu.VMEM((C * L, DP), jnp.float32),
        pltpu.VMEM((C * L, DP), jnp.float32),
        pltpu.VMEM((C, DP), jnp.float32),
        pltpu.VMEM((C, DP), jnp.float32),
        pltpu.SemaphoreType.DMA,
        pltpu.SemaphoreType.DMA,
    ]
    if with_user:
        scratch += [
            pltpu.VMEM((C,), jnp.int32),
            pltpu.VMEM((C,), jnp.int32),
            pltpu.VMEM((C, DP), jnp.float32),
            pltpu.VMEM((C, DP), jnp.float32),
            pltpu.SemaphoreType.DMA,
            pltpu.SemaphoreType.DMA,
        ]

    def body(*refs):
        if with_user:
            (item_hbm, user_hbm, sess_hbm, nodes_hbm, st_out, lt_out,
             i0, i1, r0, r1, o0, o1, sem0, sem1,
             ui0, ui1, ur0, ur1, us0, us1) = refs
            uidx_v, urows_v, usems = (ui0, ui1), (ur0, ur1), (us0, us1)
        else:
            (item_hbm, sess_hbm, st_out,
             i0, i1, r0, r1, o0, o1, sem0, sem1) = refs
        idx_v, rows_v, out_v, sems = (i0, i1), (r0, r1), (o0, o1), (sem0, sem1)
        w = lax.axis_index("s") * NC + lax.axis_index("c")
        w0 = w * steps

        def fetch(c, b):
            base = (w0 + c) * C
            pltpu.sync_copy(sess_hbm.at[pl.ds(base * L, C * L)], idx_v[b])
            pltpu.make_async_copy(
                item_hbm.at[idx_v[b]], rows_v[b], sems[b]).start()
            if with_user:
                pltpu.sync_copy(nodes_hbm.at[pl.ds(base, C)], uidx_v[b])
                pltpu.make_async_copy(
                    user_hbm.at[uidx_v[b]], urows_v[b], usems[b]).start()

        def consume(c, b):
            base = (w0 + c) * C
            pltpu.make_async_copy(
                item_hbm.at[idx_v[b]], rows_v[b], sems[b]).wait()

            def sess(s, c2):
                for k in range(DP // 16):
                    off = k * 16
                    a = rows_v[b][s * L, pl.ds(off, 16)]
                    for l in range(1, L):
                        a = a + rows_v[b][s * L + l, pl.ds(off, 16)]
                    out_v[b][s, pl.ds(off, 16)] = a
                return c2

            lax.fori_loop(0, C, sess, 0)
            pltpu.sync_copy(out_v[b], st_out.at[pl.ds(base, C)])
            if with_user:
                pltpu.make_async_copy(
                    user_hbm.at[uidx_v[b]], urows_v[b], usems[b]).wait()
                pltpu.sync_copy(urows_v[b], lt_out.at[pl.ds(base, C)])

        fetch(0, 0)

        def step(j, carry):
            c = j * 2
            fetch(c + 1, 1)
            consume(c, 0)

            @pl.when(j < steps // 2 - 1)
            def _():
                fetch(c + 2, 0)

            consume(c + 1, 1)
            return carry

        lax.fori_loop(0, steps // 2, step, 0)

    return pl.kernel(body, out_type=out_type, mesh=mesh, scratch_types=scratch)


def _masked_mean(rs, sess, e0):
    cnt0 = jnp.sum((sess == 0).astype(jnp.float32), axis=1, keepdims=True)
    den = jnp.maximum(jnp.float32(L) - cnt0, 1.0)
    return (rs[:, :D] - cnt0 * e0) / den


def _h_block(lt_ref, rs_ref, sess_ref, wa_ref, wb_ref, e0_ref, o_ref):
    st = _masked_mean(rs_ref[...], sess_ref[...], e0_ref[...])
    x = jnp.dot(lt_ref[:, :D], wa_ref[...], preferred_element_type=jnp.float32)
    x = x + jnp.dot(st, wb_ref[...], preferred_element_type=jnp.float32)
    o_ref[...] = jnp.tanh(x)


def _h_layer(lt, rs, sess, wa, wb, e0, bs):
    n = lt.shape[0]
    return pl.pallas_call(
        _h_block,
        grid=(n // bs,),
        in_specs=[
            pl.BlockSpec((bs, DP), lambda i: (i, 0)),
            pl.BlockSpec((bs, DP), lambda i: (i, 0)),
            pl.BlockSpec((bs, L), lambda i: (i, 0)),
            pl.BlockSpec((D, D), lambda i: (0, 0)),
            pl.BlockSpec((D, D), lambda i: (0, 0)),
            pl.BlockSpec((1, D), lambda i: (0, 0)),
        ],
        out_specs=pl.BlockSpec((bs, D), lambda i: (i, 0)),
        out_shape=jax.ShapeDtypeStruct((n, D), jnp.float32),
    )(lt, rs, sess, wa, wb, e0)


def _gat_math(selfv, neigh, k, wt, b):
    sn = jnp.sum(neigh * selfv[:, None, :], axis=2)
    ss = jnp.sum(selfv * selfv, axis=1, keepdims=True)
    s = jnp.concatenate([sn, ss], axis=1)
    m = jnp.max(s, axis=1, keepdims=True)
    e = jnp.exp(s - m)
    a = e / jnp.sum(e, axis=1, keepdims=True)
    ctx = jnp.sum(neigh * a[:, :k, None], axis=1) + selfv * a[:, k:k + 1]
    return jnp.maximum(
        jnp.dot(ctx, wt, preferred_element_type=jnp.float32) + b, 0.0)


def _gat0_block(h2_ref, h1_ref, w_ref, b_ref, o_ref):
    bs = h2_ref.shape[0]
    neigh = h1_ref[...].reshape(bs, S1, D)
    o_ref[...] = _gat_math(h2_ref[...], neigh, S1, w_ref[...], b_ref[...])


def _gat0(h2, h1, wt, b, bs):
    n = h2.shape[0]
    return pl.pallas_call(
        _gat0_block,
        grid=(n // bs,),
        in_specs=[
            pl.BlockSpec((bs, D), lambda i: (i, 0)),
            pl.BlockSpec((bs * S1, D), lambda i: (i, 0)),
            pl.BlockSpec((D, D), lambda i: (0, 0)),
            pl.BlockSpec((1, D), lambda i: (0, 0)),
        ],
        out_specs=pl.BlockSpec((bs, D), lambda i: (i, 0)),
        out_shape=jax.ShapeDtypeStruct((n, D), jnp.float32),
    )(h2, h1, wt, b)


def _gat1_feat_block(hu_rs_ref, ix_ref, h2a_ref, w_ref, b_ref,
                     w2a_ref, w2b_ref, e0_ref, o_ref):
    hu = _masked_mean(hu_rs_ref[...], ix_ref[...], e0_ref[...])
    neigh = h2a_ref[...].reshape(B, S2, D)
    soc = _gat_math(hu, neigh, S2, w_ref[...], b_ref[...])
    o_ref[...] = (
        jnp.dot(hu, w2a_ref[...], preferred_element_type=jnp.float32)
        + jnp.dot(soc, w2b_ref[...], preferred_element_type=jnp.float32))


def _gat1_feat(hu_rs, ix, h2a, wt, b, w2a, w2b, e0):
    return pl.pallas_call(
        _gat1_feat_block,
        out_shape=jax.ShapeDtypeStruct((B, D), jnp.float32),
    )(hu_rs, ix, h2a, wt, b, w2a, w2b, e0)


def _logits_block(feat_ref, it_ref, o_ref):
    o_ref[...] = lax.dot_general(
        feat_ref[...], it_ref[...], (((1,), (1,)), ((), ())),
        preferred_element_type=jnp.float32)


def _logits(feat, item_emb, vb):
    nv = item_emb.shape[0]
    return pl.pallas_call(
        _logits_block,
        grid=(pl.cdiv(nv, vb),),
        in_specs=[
            pl.BlockSpec((B, D), lambda i: (0, 0)),
            pl.BlockSpec((vb, D), lambda i: (i, 0)),
        ],
        out_specs=pl.BlockSpec((B, vb), lambda i: (0, i)),
        out_shape=jax.ShapeDtypeStruct((B, nv), jnp.float32),
    )(feat, item_emb)


def kernel(input_x, support_nodes_layer1, support_nodes_layer2,
           support_sessions_layer1, support_sessions_layer2,
           item_emb, user_emb, W1, W2, gat_w, gat_b):
    input_x = jnp.asarray(input_x, jnp.int32)
    sn1 = jnp.asarray(support_nodes_layer1, jnp.int32)
    sn2 = jnp.asarray(support_nodes_layer2, jnp.int32)
    ss1 = jnp.asarray(support_sessions_layer1, jnp.int32)
    ss2 = jnp.asarray(support_sessions_layer2, jnp.int32)

    item_pad = jnp.pad(item_emb, ((0, 0), (0, DP - D)))
    user_pad = jnp.pad(user_emb, ((0, 0), (0, DP - D)))
    e0 = item_emb[0:1, :]

    hu_rs = _make_sess_kernel(B, 16, False)(item_pad, input_x.reshape(-1))
    rs1, lt1 = _make_sess_kernel(B * S2 * S1, 16, True)(
        item_pad, user_pad, ss1.reshape(-1), sn1)
    rs2, lt2 = _make_sess_kernel(B * S2, 16, True)(
        item_pad, user_pad, ss2.reshape(-1), sn2)

    w1a = W1[:, :D].T
    w1b = W1[:, D:].T
    w0t = gat_w[0].T
    w1t = gat_w[1].T
    b0 = gat_b[0].reshape(1, D)
    b1 = gat_b[1].reshape(1, D)
    w2a = W2[:, :D].T
    w2b = W2[:, D:].T

    h1 = _h_layer(lt1, rs1, ss1, w1a, w1b, e0, 512)
    h2 = _h_layer(lt2, rs2, ss2, w1a, w1b, e0, 512)
    h2_agg = _gat0(h2, h1, w0t, b0, 512)
    feat = _gat1_feat(hu_rs, input_x, h2_agg, w1t, b1, w2a, w2b, e0)
    return _logits(feat, item_emb, 2048)


# speedup vs baseline: 3.7741x; 1.0083x over previous
import functools

import jax
import jax.numpy as jnp
from jax import lax
from jax.experimental import pallas as pl
from jax.experimental.pallas import tpu as pltpu
from jax.experimental.pallas import tpu_sc as plsc

NC, NS, LANES = 2, 16, 16
NW = NC * NS
D = 100
DP = 128
L = 20
S1 = 10
S2 = 5
B = 1024


def _pad_cast_block(x_ref, o_ref, obf_ref):
    x = x_ref[...]
    z = jnp.zeros((x.shape[0], DP - D), jnp.float32)
    xp = jnp.concatenate([x, z], axis=1)
    o_ref[...] = xp
    if obf_ref is not None:
        obf_ref[...] = xp.astype(jnp.bfloat16)


def _pad_cast(x, with_bf16, bs=2000):
    n = x.shape[0]
    out_shape = [jax.ShapeDtypeStruct((n, DP), jnp.float32)]
    out_specs = [pl.BlockSpec((bs, DP), lambda i: (i, 0))]
    if with_bf16:
        out_shape.append(jax.ShapeDtypeStruct((n, DP), jnp.bfloat16))
        out_specs.append(pl.BlockSpec((bs, DP), lambda i: (i, 0)))
        fn = _pad_cast_block
    else:
        fn = lambda x_ref, o_ref: _pad_cast_block(x_ref, o_ref, None)
    return pl.pallas_call(
        fn,
        grid=(n // bs,),
        in_specs=[pl.BlockSpec((bs, D), lambda i: (i, 0))],
        out_specs=out_specs,
        out_shape=out_shape,
    )(x)


@functools.lru_cache(maxsize=None)
def _make_sess_kernel(n_rows: int, chunk: int, with_user: bool):
    per_w = n_rows // NW
    steps = per_w // chunk
    assert n_rows == NW * steps * chunk and steps % 2 == 0, (n_rows, chunk)
    C = chunk
    mesh = plsc.VectorSubcoreMesh(core_axis_name="c", subcore_axis_name="s")

    st_ty = jax.ShapeDtypeStruct((n_rows, DP), jnp.float32)
    lt_ty = jax.ShapeDtypeStruct((n_rows, DP), jnp.float32)
    out_type = (st_ty, lt_ty) if with_user else st_ty

    scratch = [
        pltpu.VMEM((C * L,), jnp.int32),
        pltpu.VMEM((C * L,), jnp.int32),
        pltpu.VMEM((C * L, DP), jnp.float32),
        pltpu.VMEM((C * L, DP), jnp.float32),
        pltpu.VMEM((C, DP), jnp.float32),
        pltpu.VMEM((C, DP), jnp.float32),
        pltpu.SemaphoreType.DMA,
        pltpu.SemaphoreType.DMA,
    ]
    if with_user:
        scratch += [
            pltpu.VMEM((C,), jnp.int32),
            pltpu.VMEM((C,), jnp.int32),
            pltpu.VMEM((C, DP), jnp.float32),
            pltpu.VMEM((C, DP), jnp.float32),
            pltpu.SemaphoreType.DMA,
            pltpu.SemaphoreType.DMA,
        ]

    def body(*refs):
        if with_user:
            (item_hbm, user_hbm, sess_hbm, nodes_hbm, st_out, lt_out,
             i0, i1, r0, r1, o0, o1, sem0, sem1,
             ui0, ui1, ur0, ur1, us0, us1) = refs
            uidx_v, urows_v, usems = (ui0, ui1), (ur0, ur1), (us0, us1)
        else:
            (item_hbm, sess_hbm, st_out,
             i0, i1, r0, r1, o0, o1, sem0, sem1) = refs
        idx_v, rows_v, out_v, sems = (i0, i1), (r0, r1), (o0, o1), (sem0, sem1)
        w = lax.axis_index("s") * NC + lax.axis_index("c")
        w0 = w * steps

        def fetch(c, b):
            base = (w0 + c) * C
            pltpu.sync_copy(sess_hbm.at[pl.ds(base * L, C * L)], idx_v[b])
            pltpu.make_async_copy(
                item_hbm.at[idx_v[b]], rows_v[b], sems[b]).start()
            if with_user:
                pltpu.sync_copy(nodes_hbm.at[pl.ds(base, C)], uidx_v[b])
                pltpu.make_async_copy(
                    user_hbm.at[uidx_v[b]], urows_v[b], usems[b]).start()

        def consume(c, b):
            base = (w0 + c) * C
            pltpu.make_async_copy(
                item_hbm.at[idx_v[b]], rows_v[b], sems[b]).wait()

            def sess(s, c2):
                for k in range(DP // 16):
                    off = k * 16
                    a = rows_v[b][s * L, pl.ds(off, 16)]
                    for l in range(1, L):
                        a = a + rows_v[b][s * L + l, pl.ds(off, 16)]
                    out_v[b][s, pl.ds(off, 16)] = a
                return c2

            lax.fori_loop(0, C, sess, 0)
            pltpu.sync_copy(out_v[b], st_out.at[pl.ds(base, C)])
            if with_user:
                pltpu.make_async_copy(
                    user_hbm.at[uidx_v[b]], urows_v[b], usems[b]).wait()
                pltpu.sync_copy(urows_v[b], lt_out.at[pl.ds(base, C)])

        fetch(0, 0)

        def step(j, carry):
            c = j * 2
            fetch(c + 1, 1)
            consume(c, 0)

            @pl.when(j < steps // 2 - 1)
            def _():
                fetch(c + 2, 0)

            consume(c + 1, 1)
            return carry

        lax.fori_loop(0, steps // 2, step, 0)

    return pl.kernel(body, out_type=out_type, mesh=mesh, scratch_types=scratch)


def _interleave(sess):
    n = sess.shape[0]
    return sess.reshape(n // 2, 2, L).transpose(0, 2, 1).reshape(-1)


def _masked_mean(rs, sess, e0):
    cnt0 = jnp.sum((sess == 0).astype(jnp.float32), axis=1, keepdims=True)
    den = jnp.maximum(jnp.float32(L) - cnt0, 1.0)
    return (rs[:, :D] - cnt0 * e0) / den


def _h_block(lt_ref, rs_ref, sess_ref, wa_ref, wb_ref, e0_ref, o_ref):
    st = _masked_mean(rs_ref[...], sess_ref[...], e0_ref[...])
    lt = lt_ref[:, :D]
    x = jnp.dot(lt, wa_ref[...], preferred_element_type=jnp.float32)
    x = x + jnp.dot(st, wb_ref[...], preferred_element_type=jnp.float32)
    o_ref[...] = jnp.tanh(x)


def _h_layer(lt, rs, sess, wa, wb, e0, bs):
    n = lt.shape[0]
    return pl.pallas_call(
        _h_block,
        grid=(n // bs,),
        in_specs=[
            pl.BlockSpec((bs, DP), lambda i: (i, 0)),
            pl.BlockSpec((bs, DP), lambda i: (i, 0)),
            pl.BlockSpec((bs, L), lambda i: (i, 0)),
            pl.BlockSpec((D, D), lambda i: (0, 0)),
            pl.BlockSpec((D, D), lambda i: (0, 0)),
            pl.BlockSpec((1, D), lambda i: (0, 0)),
        ],
        out_specs=pl.BlockSpec((bs, D), lambda i: (i, 0)),
        out_shape=jax.ShapeDtypeStruct((n, D), jnp.float32),
    )(lt, rs, sess, wa, wb, e0)


def _gat_math(selfv, neigh, k, wt, b):
    sn = jnp.sum(neigh * selfv[:, None, :], axis=2)
    ss = jnp.sum(selfv * selfv, axis=1, keepdims=True)
    s = jnp.concatenate([sn, ss], axis=1)
    m = jnp.max(s, axis=1, keepdims=True)
    e = jnp.exp(s - m)
    a = e / jnp.sum(e, axis=1, keepdims=True)
    ctx = jnp.sum(neigh * a[:, :k, None], axis=1) + selfv * a[:, k:k + 1]
    return jnp.maximum(
        jnp.dot(ctx, wt, preferred_element_type=jnp.float32) + b, 0.0)


def _gat0_block(h2_ref, h1_ref, w_ref, b_ref, o_ref):
    bs = h2_ref.shape[0]
    neigh = h1_ref[...].reshape(bs, S1, D)
    o_ref[...] = _gat_math(h2_ref[...], neigh, S1, w_ref[...], b_ref[...])


def _gat0(h2, h1, wt, b, bs):
    n = h2.shape[0]
    return pl.pallas_call(
        _gat0_block,
        grid=(n // bs,),
        in_specs=[
            pl.BlockSpec((bs, D), lambda i: (i, 0)),
            pl.BlockSpec((bs * S1, D), lambda i: (i, 0)),
            pl.BlockSpec((D, D), lambda i: (0, 0)),
            pl.BlockSpec((1, D), lambda i: (0, 0)),
        ],
        out_specs=pl.BlockSpec((bs, D), lambda i: (i, 0)),
        out_shape=jax.ShapeDtypeStruct((n, D), jnp.float32),
    )(h2, h1, wt, b)


def _gat1_feat_block(hu_rs_ref, ix_ref, h2a_ref, w_ref, b_ref,
                     w2a_ref, w2b_ref, e0_ref, o_ref):
    hu = _masked_mean(hu_rs_ref[...], ix_ref[...], e0_ref[...])
    neigh = h2a_ref[...].reshape(B, S2, D)
    soc = _gat_math(hu, neigh, S2, w_ref[...], b_ref[...])
    feat = (jnp.dot(hu, w2a_ref[...], preferred_element_type=jnp.float32)
            + jnp.dot(soc, w2b_ref[...], preferred_element_type=jnp.float32))
    o_ref[...] = jnp.concatenate(
        [feat, jnp.zeros((B, DP - D), jnp.float32)], axis=1)


def _gat1_feat(hu_rs, ix, h2a, wt, b, w2a, w2b, e0):
    return pl.pallas_call(
        _gat1_feat_block,
        out_shape=jax.ShapeDtypeStruct((B, DP), jnp.float32),
    )(hu_rs, ix, h2a, wt, b, w2a, w2b, e0)


def _logits_block(feat_ref, it_ref, o_ref):
    f = feat_ref[...].astype(jnp.bfloat16)
    o_ref[...] = lax.dot_general(
        f, it_ref[...], (((1,), (1,)), ((), ())),
        preferred_element_type=jnp.float32)


def _logits(feat, item_bf, vb):
    nv = item_bf.shape[0]
    return pl.pallas_call(
        _logits_block,
        grid=(pl.cdiv(nv, vb),),
        in_specs=[
            pl.BlockSpec((B, DP), lambda i: (0, 0)),
            pl.BlockSpec((vb, DP), lambda i: (i, 0)),
        ],
        out_specs=pl.BlockSpec((B, vb), lambda i: (0, i)),
        out_shape=jax.ShapeDtypeStruct((B, nv), jnp.float32),
    )(feat, item_bf)


def kernel(input_x, support_nodes_layer1, support_nodes_layer2,
           support_sessions_layer1, support_sessions_layer2,
           item_emb, user_emb, W1, W2, gat_w, gat_b):
    input_x = jnp.asarray(input_x, jnp.int32)
    sn1 = jnp.asarray(support_nodes_layer1, jnp.int32)
    sn2 = jnp.asarray(support_nodes_layer2, jnp.int32)
    ss1 = jnp.asarray(support_sessions_layer1, jnp.int32)
    ss2 = jnp.asarray(support_sessions_layer2, jnp.int32)

    item_pad, item_bf = _pad_cast(item_emb, True)
    (user_pad,) = _pad_cast(user_emb, False)
    e0 = item_emb[0:1, :]

    hu_rs = _make_sess_kernel(B, 16, False)(item_pad, input_x.reshape(-1))
    rs1, lt1 = _make_sess_kernel(B * S2 * S1, 16, True)(
        item_pad, user_pad, ss1.reshape(-1), sn1)
    rs2, lt2 = _make_sess_kernel(B * S2, 16, True)(
        item_pad, user_pad, ss2.reshape(-1), sn2)

    w1a = W1[:, :D].T
    w1b = W1[:, D:].T
    w0t = gat_w[0].T
    w1t = gat_w[1].T
    b0 = gat_b[0].reshape(1, D)
    b1 = gat_b[1].reshape(1, D)
    w2a = W2[:, :D].T
    w2b = W2[:, D:].T

    h1 = _h_layer(lt1, rs1, ss1, w1a, w1b, e0, 512)
    h2 = _h_layer(lt2, rs2, ss2, w1a, w1b, e0, 512)
    h2_agg = _gat0(h2, h1, w0t, b0, 512)
    feat = _gat1_feat(hu_rs, input_x, h2_agg, w1t, b1, w2a, w2b, e0)
    return _logits(feat, item_bf, 2048)
